# Initial kernel scaffold; baseline (speedup 1.0000x reference)
import jax
import jax.numpy as jnp
from jax.experimental import pallas as pl


def kernel(edge_index, user_emb, item_emb):
    raise NotImplementedError("write your pallas kernel here")


# speedup vs baseline: 6.0480x; 6.0480x over previous
import functools

import jax
import jax.numpy as jnp
from jax import lax
from jax.experimental import pallas as pl
from jax.experimental.pallas import tpu as pltpu
from jax.experimental.pallas import tpu_sc as plsc

NC = 2
NS = 16
LANES = 16

EB = 64
CH = 64

N_LAYERS = 3


def _splat(ref, i):
  idx = jnp.full((LANES,), i, dtype=jnp.int32)
  return plsc.load_gather(ref, [idx])


def _rsqrt16(d):
  xi = lax.bitcast_convert_type(d, jnp.int32)
  yi = jnp.int32(0x5F3759DF) - (xi >> 1)
  y = lax.bitcast_convert_type(yi, jnp.float32)
  for _ in range(3):
    y = y * (1.5 - 0.5 * d * y * y)
  return jnp.where(d > 0.5, y, 0.0)


def _row_scale(dst, src, scale_ref, base, rows, extra=None, extra_w=0.0):
  nvec = src.shape[1] // LANES

  @pl.loop(0, rows)
  def _(i):
    s = _splat(scale_ref, base + i)
    for j in range(nvec):
      sl = pl.ds(j * LANES, LANES)
      v = src[i, sl] * s
      if extra is not None:
        v = v + extra_w * extra[i, sl]
      dst[i, sl] = v


def _build_sc_call(n_pad, n_batches, dh):
  rpt = n_pad // NS
  n_chunks = rpt // CH
  nvec = dh // LANES

  mesh = plsc.VectorSubcoreMesh(
      core_axis_name="c", subcore_axis_name="s", num_cores=NC,
      num_subcores=NS)

  @functools.partial(
      pl.kernel,
      out_type=(
          jax.ShapeDtypeStruct((NC, n_pad, dh), jnp.float32),
          jax.ShapeDtypeStruct((N_LAYERS + 1, NC, n_pad, dh), jnp.float32),
      ),
      mesh=mesh,
      compiler_params=pltpu.CompilerParams(needs_layout_passes=False),
      scratch_types=dict(
          acc_sh=pltpu.VMEM_SHARED((n_pad, dh), jnp.float32),
          deg_sh=pltpu.VMEM_SHARED((n_pad,), jnp.float32),
          rbuf=pltpu.VMEM((EB,), jnp.int32),
          cbuf=pltpu.VMEM((EB,), jnp.int32),
          gbuf=pltpu.VMEM((2, EB, dh), jnp.float32),
          work=pltpu.VMEM((CH, dh), jnp.float32),
          work2=pltpu.VMEM((CH, dh), jnp.float32),
          zdeg=pltpu.VMEM((rpt,), jnp.float32),
          degb=pltpu.VMEM((rpt,), jnp.float32),
          dinv=pltpu.VMEM((rpt,), jnp.float32),
          dinv2=pltpu.VMEM((rpt,), jnp.float32),
          qr=pltpu.VMEM((rpt,), jnp.float32),
          obuf=pltpu.VMEM((EB,), jnp.float32),
          gsem=pltpu.SemaphoreType.DMA,
      ),
  )
  def sc_call(row_hbm, col_hbm, x0_hbm, out_hbm, y_hbm,
              acc_sh, deg_sh, rbuf, cbuf, gbuf, work, work2,
              zdeg, degb, dinv, dinv2, qr, obuf, gsem):
    c = lax.axis_index("c")
    tid = lax.axis_index("s")
    row0 = tid * rpt

    zeros = jnp.zeros((LANES,), jnp.float32)

    @pl.loop(0, CH)
    def _(i):
      for j in range(nvec):
        work2[i, pl.ds(j * LANES, LANES)] = zeros

    @pl.loop(0, rpt // LANES)
    def _(i):
      zdeg[pl.ds(i * LANES, LANES)] = zeros

    @pl.loop(0, EB // LANES)
    def _(i):
      obuf[pl.ds(i * LANES, LANES)] = jnp.ones((LANES,), jnp.float32)

    pltpu.sync_copy(zdeg, deg_sh.at[pl.ds(row0, rpt)])
    plsc.subcore_barrier()

    @pl.loop(0, n_batches)
    def _(b):
      pltpu.sync_copy(col_hbm.at[tid, b, 0], cbuf)
      pltpu.sync_copy(obuf, deg_sh.at[cbuf], add=True)

    plsc.subcore_barrier()

    pltpu.sync_copy(deg_sh.at[pl.ds(row0, rpt)], degb)

    @pl.loop(0, rpt // LANES)
    def _(i):
      sl = pl.ds(i * LANES, LANES)
      d = degb[sl]
      di = _rsqrt16(d)
      dinv[sl] = di
      dinv2[sl] = di * di
      qr[sl] = 0.25 * d * di

    for k in range(n_chunks):
      nsl = pl.ds(row0 + k * CH, CH)
      pltpu.sync_copy(x0_hbm.at[c].at[nsl], work)
      _row_scale(work, work, dinv, k * CH, CH)
      pltpu.sync_copy(work, y_hbm.at[0].at[c].at[nsl])
    plsc.subcore_barrier()

    for layer in range(1, N_LAYERS + 1):
      for k in range(n_chunks):
        pltpu.sync_copy(work2, acc_sh.at[pl.ds(row0 + k * CH, CH)])
      plsc.subcore_barrier()

      ysrc = y_hbm.at[layer - 1].at[c]

      def _gather(slot):
        pltpu.make_async_copy(ysrc.at[rbuf], gbuf.at[slot], gsem).start()

      def _gather_wait(slot):
        pltpu.make_async_copy(ysrc.at[rbuf], gbuf.at[slot], gsem).wait()

      pltpu.sync_copy(row_hbm.at[tid, 0, 0], rbuf)
      pltpu.sync_copy(col_hbm.at[tid, 0, 0], cbuf)
      _gather(0)

      @pl.loop(0, n_batches)
      def _(b):
        slot = lax.rem(b, 2)
        _gather_wait(slot)

        @pl.when(b < n_batches - 1)
        def _():
          pltpu.sync_copy(row_hbm.at[tid, b + 1, 0], rbuf)
          _gather(1 - slot)

        pltpu.sync_copy(gbuf.at[slot], acc_sh.at[cbuf], add=True)

        @pl.when(b < n_batches - 1)
        def _():
          pltpu.sync_copy(col_hbm.at[tid, b + 1, 0], cbuf)

      plsc.subcore_barrier()

      for k in range(n_chunks):
        nsl = pl.ds(row0 + k * CH, CH)
        pltpu.sync_copy(acc_sh.at[nsl], work)
        _row_scale(work, work, dinv2, k * CH, CH)
        pltpu.sync_copy(work, y_hbm.at[layer].at[c].at[nsl])
      plsc.subcore_barrier()

    for k in range(n_chunks):
      nsl = pl.ds(row0 + k * CH, CH)
      pltpu.sync_copy(y_hbm.at[1].at[c].at[nsl], work)
      for layer in range(2, N_LAYERS + 1):
        pltpu.sync_copy(y_hbm.at[layer].at[c].at[nsl], work2)

        @pl.loop(0, CH)
        def _(i):
          for j in range(nvec):
            sl = pl.ds(j * LANES, LANES)
            work[i, sl] = work[i, sl] + work2[i, sl]

      pltpu.sync_copy(x0_hbm.at[c].at[nsl], work2)
      _row_scale(work, work, qr, k * CH, CH, extra=work2, extra_w=0.25)
      pltpu.sync_copy(work, out_hbm.at[c].at[nsl])

  return sc_call


def kernel(edge_index, user_emb, item_emb):
  n_users = user_emb.shape[0]
  n = n_users + item_emb.shape[0]
  d = user_emb.shape[1]
  dh = d // NC
  e = edge_index.shape[1]

  n_pad = ((n + NS * 128 - 1) // (NS * 128)) * (NS * 128)
  e_pad = ((e + NS * EB - 1) // (NS * EB)) * NS * EB
  n_batches = e_pad // (NS * EB)

  all_emb = jnp.concatenate([user_emb, item_emb], axis=0)
  x0 = jnp.pad(all_emb, ((0, n_pad - n), (0, 0)))
  x0 = x0.reshape(n_pad, NC, dh).transpose(1, 0, 2)

  row = jnp.pad(edge_index[0].astype(jnp.int32), (0, e_pad - e),
                constant_values=n_pad - 1).reshape(NS, -1, 1, EB)
  col = jnp.pad(edge_index[1].astype(jnp.int32), (0, e_pad - e),
                constant_values=n_pad - 1).reshape(NS, -1, 1, EB)

  sc_call = _build_sc_call(n_pad, n_batches, dh)
  out, _ = sc_call(row, col, x0)

  final = out.transpose(1, 0, 2).reshape(n_pad, d)[:n]
  return final[:n_users], final[n_users:]


# speedup vs baseline: 9.7347x; 1.6096x over previous
import functools

import jax
import jax.numpy as jnp
from jax import lax
from jax.experimental import pallas as pl
from jax.experimental.pallas import tpu as pltpu
from jax.experimental.pallas import tpu_sc as plsc

NC = 2
NS = 16
LANES = 16

EB = 128
CH = 128

N_LAYERS = 3


def _splat(ref, i):
  idx = jnp.full((LANES,), i, dtype=jnp.int32)
  return plsc.load_gather(ref, [idx])


def _rsqrt16(d):
  xi = lax.bitcast_convert_type(d, jnp.int32)
  yi = jnp.int32(0x5F3759DF) - (xi >> 1)
  y = lax.bitcast_convert_type(yi, jnp.float32)
  for _ in range(3):
    y = y * (1.5 - 0.5 * d * y * y)
  return jnp.where(d > 0.5, y, 0.0)


def _build_sc_call(n_pad, n_batches, dh):
  rpt = n_pad // NS
  n_chunks = rpt // CH
  nvec = dh // LANES
  nb = n_batches

  mesh = plsc.VectorSubcoreMesh(
      core_axis_name="c", subcore_axis_name="s", num_cores=NC,
      num_subcores=NS)

  @functools.partial(
      pl.kernel,
      out_type=(
          jax.ShapeDtypeStruct((NC, n_pad, dh), jnp.float32),
          jax.ShapeDtypeStruct((N_LAYERS + 1, NC, n_pad, dh), jnp.float32),
      ),
      mesh=mesh,
      compiler_params=pltpu.CompilerParams(needs_layout_passes=False),
      scratch_types=dict(
          acc_sh=pltpu.VMEM_SHARED((n_pad, dh), jnp.float32),
          deg_sh=pltpu.VMEM_SHARED((n_pad,), jnp.float32),
          rbuf=pltpu.VMEM((3, EB), jnp.int32),
          cbuf=pltpu.VMEM((3, EB), jnp.int32),
          gbuf=pltpu.VMEM((2, EB, dh), jnp.float32),
          zdeg=pltpu.VMEM((rpt,), jnp.float32),
          degb=pltpu.VMEM((rpt,), jnp.float32),
          dinv=pltpu.VMEM((rpt,), jnp.float32),
          dinv2=pltpu.VMEM((rpt,), jnp.float32),
          qr=pltpu.VMEM((rpt,), jnp.float32),
          obuf=pltpu.VMEM((EB,), jnp.float32),
          gsem=pltpu.SemaphoreType.DMA,
          isem=pltpu.SemaphoreType.DMA,
      ),
  )
  def sc_call(row_hbm, col_hbm, x0_hbm, out_hbm, y_hbm,
              acc_sh, deg_sh, rbuf, cbuf, gbuf, zdeg, degb, dinv, dinv2,
              qr, obuf, gsem, isem):
    c = lax.axis_index("c")
    tid = lax.axis_index("s")
    row0 = tid * rpt
    zeros = jnp.zeros((LANES,), jnp.float32)

    def scale3(lead, scale_ref, base, rows, extra_lead=None, extra_w=0.0):
      @pl.loop(0, rows)
      def _(i):
        s = _splat(scale_ref, base + i)
        for j in range(nvec):
          sl = pl.ds(j * LANES, LANES)
          v = gbuf[lead, i, sl] * s
          if extra_lead is not None:
            v = v + extra_w * gbuf[extra_lead, i, sl]
          gbuf[lead, i, sl] = v

    def zero_gbuf(lead):
      @pl.loop(0, EB)
      def _(i):
        for j in range(nvec):
          gbuf[lead, i, pl.ds(j * LANES, LANES)] = zeros

    def idx_start(b, islot):
      pltpu.make_async_copy(row_hbm.at[tid, b, 0], rbuf.at[islot], isem).start()
      pltpu.make_async_copy(col_hbm.at[tid, b, 0], cbuf.at[islot], isem).start()

    def idx_wait(b, islot):
      pltpu.make_async_copy(row_hbm.at[tid, b, 0], rbuf.at[islot], isem).wait()
      pltpu.make_async_copy(col_hbm.at[tid, b, 0], cbuf.at[islot], isem).wait()

    @pl.loop(0, rpt // LANES)
    def _(i):
      zdeg[pl.ds(i * LANES, LANES)] = zeros

    @pl.loop(0, EB // LANES)
    def _(i):
      obuf[pl.ds(i * LANES, LANES)] = jnp.ones((LANES,), jnp.float32)

    pltpu.sync_copy(zdeg, deg_sh.at[pl.ds(row0, rpt)])
    plsc.subcore_barrier()

    pltpu.sync_copy(col_hbm.at[tid, 0, 0], cbuf.at[0])
    pltpu.make_async_copy(col_hbm.at[tid, 1, 0], cbuf.at[1], isem).start()

    @pl.loop(0, nb)
    def _(b):
      islot = lax.rem(b, 3)

      @pl.when(b < nb - 2)
      def _():
        pltpu.make_async_copy(
            col_hbm.at[tid, b + 2, 0], cbuf.at[lax.rem(b + 2, 3)],
            isem).start()

      pltpu.sync_copy(obuf, deg_sh.at[cbuf.at[islot]], add=True)

      @pl.when(b < nb - 1)
      def _():
        pltpu.make_async_copy(
            col_hbm.at[tid, b + 1, 0], cbuf.at[lax.rem(b + 1, 3)],
            isem).wait()

    plsc.subcore_barrier()

    pltpu.sync_copy(deg_sh.at[pl.ds(row0, rpt)], degb)

    @pl.loop(0, rpt // LANES)
    def _(i):
      sl = pl.ds(i * LANES, LANES)
      d = degb[sl]
      di = _rsqrt16(d)
      dinv[sl] = di
      dinv2[sl] = di * di
      qr[sl] = 0.25 * d * di

    for k in range(n_chunks):
      nsl = pl.ds(row0 + k * CH, CH)
      pltpu.sync_copy(x0_hbm.at[c].at[nsl], gbuf.at[0])
      scale3(0, dinv, k * CH, CH)
      pltpu.sync_copy(gbuf.at[0], y_hbm.at[0].at[c].at[nsl])
    plsc.subcore_barrier()

    for layer in range(1, N_LAYERS + 1):
      zero_gbuf(0)
      for k in range(n_chunks):
        pltpu.sync_copy(gbuf.at[0], acc_sh.at[pl.ds(row0 + k * CH, CH)])
      plsc.subcore_barrier()

      ysrc = y_hbm.at[layer - 1].at[c]

      def _gather(islot, slot, ysrc=ysrc):
        pltpu.make_async_copy(
            ysrc.at[rbuf.at[islot]], gbuf.at[slot], gsem).start()

      def _gather_wait(islot, slot, ysrc=ysrc):
        pltpu.make_async_copy(
            ysrc.at[rbuf.at[islot]], gbuf.at[slot], gsem).wait()

      pltpu.sync_copy(row_hbm.at[tid, 0, 0], rbuf.at[0])
      pltpu.sync_copy(col_hbm.at[tid, 0, 0], cbuf.at[0])
      _gather(0, 0)
      idx_start(1, 1)

      @pl.loop(0, nb)
      def _(b):
        slot = lax.rem(b, 2)
        islot = lax.rem(b, 3)
        _gather_wait(islot, slot)

        @pl.when(b < nb - 1)
        def _():
          idx_wait(b + 1, lax.rem(b + 1, 3))
          _gather(lax.rem(b + 1, 3), 1 - slot)

        @pl.when(b < nb - 2)
        def _():
          idx_start(b + 2, lax.rem(b + 2, 3))

        pltpu.sync_copy(gbuf.at[slot], acc_sh.at[cbuf.at[islot]], add=True)

      plsc.subcore_barrier()

      for k in range(n_chunks):
        nsl = pl.ds(row0 + k * CH, CH)
        pltpu.sync_copy(acc_sh.at[nsl], gbuf.at[0])
        scale3(0, dinv2, k * CH, CH)
        pltpu.sync_copy(gbuf.at[0], y_hbm.at[layer].at[c].at[nsl])
      plsc.subcore_barrier()

    for k in range(n_chunks):
      nsl = pl.ds(row0 + k * CH, CH)
      pltpu.sync_copy(y_hbm.at[1].at[c].at[nsl], gbuf.at[0])
      for layer in range(2, N_LAYERS + 1):
        pltpu.sync_copy(y_hbm.at[layer].at[c].at[nsl], gbuf.at[1])

        @pl.loop(0, CH)
        def _(i):
          for j in range(nvec):
            sl = pl.ds(j * LANES, LANES)
            gbuf[0, i, sl] = gbuf[0, i, sl] + gbuf[1, i, sl]

      pltpu.sync_copy(x0_hbm.at[c].at[nsl], gbuf.at[1])
      scale3(0, qr, k * CH, CH, extra_lead=1, extra_w=0.25)
      pltpu.sync_copy(gbuf.at[0], out_hbm.at[c].at[nsl])

  return sc_call


def kernel(edge_index, user_emb, item_emb):
  n_users = user_emb.shape[0]
  n = n_users + item_emb.shape[0]
  d = user_emb.shape[1]
  dh = d // NC
  e = edge_index.shape[1]

  n_pad = ((n + NS * 128 - 1) // (NS * 128)) * (NS * 128)
  e_pad = ((e + NS * EB - 1) // (NS * EB)) * NS * EB
  n_batches = e_pad // (NS * EB)

  all_emb = jnp.concatenate([user_emb, item_emb], axis=0)
  x0 = jnp.pad(all_emb, ((0, n_pad - n), (0, 0)))
  x0 = x0.reshape(n_pad, NC, dh).transpose(1, 0, 2)

  row = jnp.pad(edge_index[0].astype(jnp.int32), (0, e_pad - e),
                constant_values=n_pad - 1).reshape(NS, -1, 1, EB)
  col = jnp.pad(edge_index[1].astype(jnp.int32), (0, e_pad - e),
                constant_values=n_pad - 1).reshape(NS, -1, 1, EB)

  sc_call = _build_sc_call(n_pad, n_batches, dh)
  out, _ = sc_call(row, col, x0)

  final = out.transpose(1, 0, 2).reshape(n_pad, d)[:n]
  return final[:n_users], final[n_users:]


# speedup vs baseline: 10.0573x; 1.0331x over previous
import functools

import jax
import jax.numpy as jnp
from jax import lax
from jax.experimental import pallas as pl
from jax.experimental.pallas import tpu as pltpu
from jax.experimental.pallas import tpu_sc as plsc

NC = 2
NS = 16
LANES = 16

EB = 128
CH = 128

N_LAYERS = 3


def _splat(ref, i):
  idx = jnp.full((LANES,), i, dtype=jnp.int32)
  return plsc.load_gather(ref, [idx])


def _rsqrt16(d):
  xi = lax.bitcast_convert_type(d, jnp.int32)
  yi = jnp.int32(0x5F3759DF) - (xi >> 1)
  y = lax.bitcast_convert_type(yi, jnp.float32)
  for _ in range(3):
    y = y * (1.5 - 0.5 * d * y * y)
  return jnp.where(d > 0.5, y, 0.0)


def _build_sc_call(n_pad, n_batches, dh):
  rpt = n_pad // NS
  n_chunks = rpt // CH
  nvec = dh // LANES
  nb = n_batches

  mesh = plsc.VectorSubcoreMesh(
      core_axis_name="c", subcore_axis_name="s", num_cores=NC,
      num_subcores=NS)

  @functools.partial(
      pl.kernel,
      out_type=(
          jax.ShapeDtypeStruct((NC, n_pad, dh), jnp.float32),
          jax.ShapeDtypeStruct((N_LAYERS + 1, NC, n_pad, dh), jnp.float32),
      ),
      mesh=mesh,
      compiler_params=pltpu.CompilerParams(needs_layout_passes=False),
      scratch_types=dict(
          acc_sh=pltpu.VMEM_SHARED((n_pad, dh), jnp.float32),
          deg_sh=pltpu.VMEM_SHARED((n_pad,), jnp.float32),
          rbuf=pltpu.VMEM((3, EB), jnp.int32),
          cbuf=pltpu.VMEM((3, EB), jnp.int32),
          gbuf=pltpu.VMEM((2, EB, dh), jnp.float32),
          zdeg=pltpu.VMEM((rpt,), jnp.float32),
          degb=pltpu.VMEM((rpt,), jnp.float32),
          dinv=pltpu.VMEM((rpt,), jnp.float32),
          dinv2=pltpu.VMEM((rpt,), jnp.float32),
          qr=pltpu.VMEM((rpt,), jnp.float32),
          obuf=pltpu.VMEM((EB,), jnp.float32),
          gsem=pltpu.SemaphoreType.DMA,
          isem=pltpu.SemaphoreType.DMA,
          ssem=pltpu.SemaphoreType.DMA,
      ),
  )
  def sc_call(row_hbm, col_hbm, x0_hbm, out_hbm, y_hbm,
              acc_sh, deg_sh, rbuf, cbuf, gbuf, zdeg, degb, dinv, dinv2,
              qr, obuf, gsem, isem, ssem):
    c = lax.axis_index("c")
    tid = lax.axis_index("s")
    row0 = tid * rpt
    zeros = jnp.zeros((LANES,), jnp.float32)

    def scale3(lead, scale_ref, base, rows, extra_lead=None, extra_w=0.0):
      @pl.loop(0, rows)
      def _(i):
        s = _splat(scale_ref, base + i)
        for j in range(nvec):
          sl = pl.ds(j * LANES, LANES)
          v = gbuf[lead, i, sl] * s
          if extra_lead is not None:
            v = v + extra_w * gbuf[extra_lead, i, sl]
          gbuf[lead, i, sl] = v

    def zero_gbuf(lead):
      @pl.loop(0, EB)
      def _(i):
        for j in range(nvec):
          gbuf[lead, i, pl.ds(j * LANES, LANES)] = zeros

    def idx_start(b, islot):
      pltpu.make_async_copy(row_hbm.at[tid, b, 0], rbuf.at[islot], isem).start()
      pltpu.make_async_copy(col_hbm.at[tid, b, 0], cbuf.at[islot], isem).start()

    def idx_wait(b, islot):
      pltpu.make_async_copy(row_hbm.at[tid, b, 0], rbuf.at[islot], isem).wait()
      pltpu.make_async_copy(col_hbm.at[tid, b, 0], cbuf.at[islot], isem).wait()

    @pl.loop(0, rpt // LANES)
    def _(i):
      zdeg[pl.ds(i * LANES, LANES)] = zeros

    @pl.loop(0, EB // LANES)
    def _(i):
      obuf[pl.ds(i * LANES, LANES)] = jnp.ones((LANES,), jnp.float32)

    pltpu.sync_copy(zdeg, deg_sh.at[pl.ds(row0, rpt)])
    plsc.subcore_barrier()

    pltpu.sync_copy(col_hbm.at[tid, 0, 0], cbuf.at[0])
    pltpu.make_async_copy(col_hbm.at[tid, 1, 0], cbuf.at[1], isem).start()

    def _hist_add(islot):
      return pltpu.make_async_copy(obuf, deg_sh.at[cbuf.at[islot]], ssem)

    @pl.loop(0, nb)
    def _(b):
      islot = lax.rem(b, 3)
      _hist_add(islot).start()

      @pl.when(b > 0)
      def _():
        _hist_add(lax.rem(b - 1, 3)).wait()

      @pl.when(b < nb - 2)
      def _():
        pltpu.make_async_copy(
            col_hbm.at[tid, b + 2, 0], cbuf.at[lax.rem(b + 2, 3)],
            isem).start()

      @pl.when(b < nb - 1)
      def _():
        pltpu.make_async_copy(
            col_hbm.at[tid, b + 1, 0], cbuf.at[lax.rem(b + 1, 3)],
            isem).wait()

    _hist_add(lax.rem(nb - 1, 3)).wait()
    plsc.subcore_barrier()

    pltpu.sync_copy(deg_sh.at[pl.ds(row0, rpt)], degb)

    @pl.loop(0, rpt // LANES)
    def _(i):
      sl = pl.ds(i * LANES, LANES)
      d = degb[sl]
      di = _rsqrt16(d)
      dinv[sl] = di
      dinv2[sl] = di * di
      qr[sl] = 0.25 * d * di

    for k in range(n_chunks):
      nsl = pl.ds(row0 + k * CH, CH)
      pltpu.sync_copy(x0_hbm.at[c].at[nsl], gbuf.at[0])
      scale3(0, dinv, k * CH, CH)
      pltpu.sync_copy(gbuf.at[0], y_hbm.at[0].at[c].at[nsl])
    plsc.subcore_barrier()

    for layer in range(1, N_LAYERS + 1):
      zero_gbuf(0)
      for k in range(n_chunks):
        pltpu.sync_copy(gbuf.at[0], acc_sh.at[pl.ds(row0 + k * CH, CH)])
      plsc.subcore_barrier()

      ysrc = y_hbm.at[layer - 1].at[c]
      half = EB // 2

      def _gather_descs(islot, slot, ysrc=ysrc):
        return (
            pltpu.make_async_copy(
                ysrc.at[rbuf.at[islot].at[pl.ds(0, half)]],
                gbuf.at[slot].at[pl.ds(0, half)], gsem),
            pltpu.make_async_copy(
                ysrc.at[rbuf.at[islot].at[pl.ds(half, half)]],
                gbuf.at[slot].at[pl.ds(half, half)], gsem),
        )

      def _gather(islot, slot):
        for d in _gather_descs(islot, slot):
          d.start()

      def _gather_wait(islot, slot):
        for d in _gather_descs(islot, slot):
          d.wait()

      def _scatter(islot, slot):
        return pltpu.make_async_copy(
            gbuf.at[slot], acc_sh.at[cbuf.at[islot]], ssem)

      pltpu.sync_copy(row_hbm.at[tid, 0, 0], rbuf.at[0])
      pltpu.sync_copy(col_hbm.at[tid, 0, 0], cbuf.at[0])
      _gather(0, 0)
      idx_start(1, 1)

      @pl.loop(0, nb)
      def _(b):
        slot = lax.rem(b, 2)
        islot = lax.rem(b, 3)
        _gather_wait(islot, slot)

        @pl.when(b < nb - 1)
        def _():
          idx_wait(b + 1, lax.rem(b + 1, 3))

        @pl.when(b > 0)
        def _():
          _scatter(lax.rem(b - 1, 3), 1 - slot).wait()

        @pl.when(b < nb - 1)
        def _():
          _gather(lax.rem(b + 1, 3), 1 - slot)

        @pl.when(b < nb - 2)
        def _():
          idx_start(b + 2, lax.rem(b + 2, 3))

        _scatter(islot, slot).start()

      _scatter(lax.rem(nb - 1, 3), lax.rem(nb - 1, 2)).wait()
      plsc.subcore_barrier()

      for k in range(n_chunks):
        nsl = pl.ds(row0 + k * CH, CH)
        pltpu.sync_copy(acc_sh.at[nsl], gbuf.at[0])
        scale3(0, dinv2, k * CH, CH)
        pltpu.sync_copy(gbuf.at[0], y_hbm.at[layer].at[c].at[nsl])
      plsc.subcore_barrier()

    for k in range(n_chunks):
      nsl = pl.ds(row0 + k * CH, CH)
      pltpu.sync_copy(y_hbm.at[1].at[c].at[nsl], gbuf.at[0])
      for layer in range(2, N_LAYERS + 1):
        pltpu.sync_copy(y_hbm.at[layer].at[c].at[nsl], gbuf.at[1])

        @pl.loop(0, CH)
        def _(i):
          for j in range(nvec):
            sl = pl.ds(j * LANES, LANES)
            gbuf[0, i, sl] = gbuf[0, i, sl] + gbuf[1, i, sl]

      pltpu.sync_copy(x0_hbm.at[c].at[nsl], gbuf.at[1])
      scale3(0, qr, k * CH, CH, extra_lead=1, extra_w=0.25)
      pltpu.sync_copy(gbuf.at[0], out_hbm.at[c].at[nsl])

  return sc_call


def kernel(edge_index, user_emb, item_emb):
  n_users = user_emb.shape[0]
  n = n_users + item_emb.shape[0]
  d = user_emb.shape[1]
  dh = d // NC
  e = edge_index.shape[1]

  n_pad = ((n + NS * 128 - 1) // (NS * 128)) * (NS * 128)
  e_pad = ((e + NS * EB - 1) // (NS * EB)) * NS * EB
  n_batches = e_pad // (NS * EB)

  all_emb = jnp.concatenate([user_emb, item_emb], axis=0)
  x0 = jnp.pad(all_emb, ((0, n_pad - n), (0, 0)))
  x0 = x0.reshape(n_pad, NC, dh).transpose(1, 0, 2)

  row = jnp.pad(edge_index[0].astype(jnp.int32), (0, e_pad - e),
                constant_values=n_pad - 1).reshape(NS, -1, 1, EB)
  col = jnp.pad(edge_index[1].astype(jnp.int32), (0, e_pad - e),
                constant_values=n_pad - 1).reshape(NS, -1, 1, EB)

  sc_call = _build_sc_call(n_pad, n_batches, dh)
  out, _ = sc_call(row, col, x0)

  final = out.transpose(1, 0, 2).reshape(n_pad, d)[:n]
  return final[:n_users], final[n_users:]
